# baseline (speedup 1.0000x reference)
import jax
import jax.numpy as jnp
from jax.experimental import pallas as pl


_COL_START = 26
_COL_COUNT = 100


def _body(in_ref, out_ref):
    out_ref[...] = in_ref[:, _COL_START:_COL_START + _COL_COUNT]


def kernel(inputs):
    n_rows, n_cols = inputs.shape
    block = 512
    return pl.pallas_call(
        _body,
        grid=(n_rows // block,),
        in_specs=[pl.BlockSpec((block, n_cols), lambda i: (i, 0))],
        out_specs=pl.BlockSpec((block, _COL_COUNT), lambda i: (i, 0)),
        out_shape=jax.ShapeDtypeStruct((n_rows, _COL_COUNT), jnp.float32),
    )(inputs)


# speedup vs baseline: 1.8007x; 1.8007x over previous
import jax
import jax.numpy as jnp
from jax.experimental import pallas as pl


_COL_START = 26
_COL_COUNT = 100


def _body(in_ref, out_ref):
    out_ref[...] = in_ref[:, _COL_START:_COL_START + _COL_COUNT]


def kernel(inputs):
    n_rows, n_cols = inputs.shape
    block = 4096
    return pl.pallas_call(
        _body,
        grid=(n_rows // block,),
        in_specs=[pl.BlockSpec((block, n_cols), lambda i: (i, 0))],
        out_specs=pl.BlockSpec((block, _COL_COUNT), lambda i: (i, 0)),
        out_shape=jax.ShapeDtypeStruct((n_rows, _COL_COUNT), jnp.float32),
    )(inputs)


# speedup vs baseline: 1.9736x; 1.0960x over previous
import jax
import jax.numpy as jnp
from jax.experimental import pallas as pl


_COL_START = 26
_COL_COUNT = 100


def _body(in_ref, out_ref):
    out_ref[...] = in_ref[:, _COL_START:_COL_START + _COL_COUNT]


def kernel(inputs):
    n_rows, n_cols = inputs.shape
    block = 8192
    return pl.pallas_call(
        _body,
        grid=(n_rows // block,),
        in_specs=[pl.BlockSpec((block, n_cols), lambda i: (i, 0))],
        out_specs=pl.BlockSpec((block, _COL_COUNT), lambda i: (i, 0)),
        out_shape=jax.ShapeDtypeStruct((n_rows, _COL_COUNT), jnp.float32),
    )(inputs)
